# Initial kernel scaffold; baseline (speedup 1.0000x reference)
import functools

import jax
import jax.numpy as jnp
from jax import lax
from jax.experimental import pallas as pl
from jax.experimental.pallas import tpu as pltpu
from jax.experimental.pallas import tpu_sc as plsc

B = 16384
SIZE = 64
M = 5
NEG = 5
NSLOT = 1 + NEG

NC = 2
NS = 16
NW = NC * NS
L = 16

ROWS_PER_W = B // NW
C = 64
NCHUNK = ROWS_PER_W // C
IDXG = 64
GPC = C * M // IDXG
NQ = SIZE // L


def _sc_body(w2m_hbm, wmask_hbm, c2m_hbm, cmask_hbm, emb0_hbm, emb1_hbm,
             out_hbm,
             widx_v, wmask_v, wrows_v, cidx_v, cmask_v, crows_v, wemb_v,
             ips_v, sem):
    wid = lax.axis_index("s") * NC + lax.axis_index("c")

    def chunk_body(ch, carry):
        base = (wid * NCHUNK + ch) * C
        ib = base * M
        gb = ib // IDXG

        pltpu.sync_copy(w2m_hbm.at[pl.ds(gb, GPC)], widx_v)
        pltpu.sync_copy(wmask_hbm.at[pl.ds(ib, C * M)], wmask_v)
        cps = [pltpu.async_copy(emb0_hbm.at[widx_v.at[g]],
                                wrows_v.at[pl.ds(g * IDXG, IDXG)], sem)
               for g in range(GPC)]
        for cp in cps:
            cp.wait()

        def wrow(r, c2):
            i0 = r * M
            acc = [jnp.zeros((L,), jnp.float32) for _ in range(NQ)]
            for m in range(M):
                wm = wmask_v[i0 + m]
                for q in range(NQ):
                    acc[q] = acc[q] + wm * wrows_v[i0 + m, pl.ds(q * L, L)]
            for q in range(NQ):
                wemb_v[r, pl.ds(q * L, L)] = acc[q]
            return c2

        lax.fori_loop(0, C, wrow, 0)

        for j in range(NSLOT):
            pltpu.sync_copy(c2m_hbm.at[j, pl.ds(gb, GPC)], cidx_v)
            pltpu.sync_copy(cmask_hbm.at[j, pl.ds(ib, C * M)], cmask_v)
            cps = [pltpu.async_copy(emb1_hbm.at[cidx_v.at[g]],
                                    crows_v.at[pl.ds(g * IDXG, IDXG)], sem)
                   for g in range(GPC)]
            for cp in cps:
                cp.wait()

            def crow(r, c2, j=j):
                i0 = r * M
                acc = [jnp.zeros((L,), jnp.float32) for _ in range(NQ)]
                for m in range(M):
                    cm = cmask_v[i0 + m]
                    for q in range(NQ):
                        acc[q] = acc[q] + cm * crows_v[i0 + m, pl.ds(q * L, L)]
                dot = jnp.zeros((L,), jnp.float32)
                for q in range(NQ):
                    dot = dot + acc[q] * wemb_v[r, pl.ds(q * L, L)]
                s = jnp.sum(dot)
                ips_v[r * 8 + j] = -s if j == 0 else s
                if j == 0:
                    ips_v[r * 8 + 6] = 0.0
                    ips_v[r * 8 + 7] = 0.0
                return c2

            lax.fori_loop(0, C, crow, 0)

        pltpu.sync_copy(ips_v, out_hbm.at[pl.ds(base * 8, C * 8)])
        return carry

    lax.fori_loop(0, NCHUNK, chunk_body, 0)


_sc_ips = pl.kernel(
    _sc_body,
    out_type=jax.ShapeDtypeStruct((B * 8,), jnp.float32),
    mesh=plsc.VectorSubcoreMesh(core_axis_name="c", subcore_axis_name="s"),
    scratch_types=[
        pltpu.VMEM((GPC, IDXG), jnp.int32),
        pltpu.VMEM((C * M,), jnp.float32),
        pltpu.VMEM((C * M, SIZE), jnp.float32),
        pltpu.VMEM((GPC, IDXG), jnp.int32),
        pltpu.VMEM((C * M,), jnp.float32),
        pltpu.VMEM((C * M, SIZE), jnp.float32),
        pltpu.VMEM((C, SIZE), jnp.float32),
        pltpu.VMEM((C * 8,), jnp.float32),
        pltpu.SemaphoreType.DMA,
    ],
)


def _loss_body(x_ref, w_ref, o_ref):
    x = jnp.clip(x_ref[...], -10.0, 10.0)
    o_ref[...] = jnp.sum(w_ref[...] * jax.nn.softplus(x), keepdims=True).reshape(1, 1)


def _loss_tc(x2d, w2d):
    return pl.pallas_call(
        _loss_body,
        out_shape=jax.ShapeDtypeStruct((1, 1), jnp.float32),
    )(x2d, w2d)


def kernel(data, word2morph, word2morph_mask, ctx2morph, ctx2morph_mask, emb0, emb1):
    w2m_g = word2morph.reshape(B * M // IDXG, IDXG)
    wmask = word2morph_mask.reshape(B * M)
    c2m_g = jnp.transpose(ctx2morph, (1, 0, 2)).reshape(NSLOT, B * M // IDXG, IDXG)
    cmask = jnp.transpose(ctx2morph_mask[..., 0], (1, 0, 2)).reshape(NSLOT, B * M)

    ips = _sc_ips(w2m_g, wmask, c2m_g, cmask, emb0, emb1)

    neg_mask = data[:, 2 + NEG:].astype(jnp.float32)
    wts = jnp.concatenate(
        [jnp.ones((B, 1), jnp.float32), neg_mask, jnp.zeros((B, 2), jnp.float32)],
        axis=1)

    loss = _loss_tc(ips.reshape(B * 8 // 128, 128), wts.reshape(B * 8 // 128, 128))
    return loss[0, 0]


# speedup vs baseline: 4.6124x; 4.6124x over previous
import jax
import jax.numpy as jnp
from jax import lax
from jax.experimental import pallas as pl
from jax.experimental.pallas import tpu as pltpu
from jax.experimental.pallas import tpu_sc as plsc

B = 16384
SIZE = 64
M = 5
NEG = 5
NSLOT = 1 + NEG

NC = 2
NS = 16
NW = NC * NS
L = 16

ROWS_PER_W = B // NW
C = 64
NCHUNK = ROWS_PER_W // C
IDXG = 64
GPC = C * M // IDXG
NQ = SIZE // L

TC_ROWS = 2048


def _sc_body(w2m_hbm, wmask_hbm, c2m_hbm, cmask_hbm, emb0_hbm, emb1_hbm,
             out_hbm,
             widx_v, wmask_v, wrows_v, cidx_v, cmask_v, crows_v, wemb_v,
             ips_v, sem):
    wid = lax.axis_index("s") * NC + lax.axis_index("c")
    zeros = jnp.zeros((L,), jnp.float32)

    def chunk_body(ch, carry):
        kb = wid * NCHUNK + ch
        base = kb * C
        ib = base * M

        pltpu.sync_copy(w2m_hbm.at[kb], widx_v)
        pltpu.sync_copy(wmask_hbm.at[pl.ds(ib, C * M)],
                        wmask_v.at[pl.ds(0, C * M)])
        cps = [pltpu.async_copy(emb0_hbm.at[widx_v.at[g]],
                                wrows_v.at[pl.ds(g * IDXG, IDXG)], sem)
               for g in range(GPC)]
        for cp in cps:
            cp.wait()

        def wrow(r, c2):
            i0 = r * M
            mvec = wmask_v[pl.ds(i0, L)]
            acc = [zeros for _ in range(NQ)]
            for m in range(M):
                wm = mvec[m]
                for q in range(NQ):
                    acc[q] = acc[q] + wm * wrows_v[i0 + m, pl.ds(q * L, L)]
            for q in range(NQ):
                wemb_v[r, pl.ds(q * L, L)] = acc[q]
            return c2

        lax.fori_loop(0, C, wrow, 0)

        for j in range(NSLOT):
            pltpu.sync_copy(c2m_hbm.at[j * NW * NCHUNK + kb], cidx_v)
            pltpu.sync_copy(cmask_hbm.at[pl.ds(j * B * M + ib, C * M)],
                            cmask_v.at[pl.ds(0, C * M)])
            cps = [pltpu.async_copy(emb1_hbm.at[cidx_v.at[g]],
                                    crows_v.at[pl.ds(g * IDXG, IDXG)], sem)
                   for g in range(GPC)]
            for cp in cps:
                cp.wait()

            def crow(r, c2, j=j):
                i0 = r * M
                mvec = cmask_v[pl.ds(i0, L)]
                acc = [zeros for _ in range(NQ)]
                for m in range(M):
                    cm = mvec[m]
                    for q in range(NQ):
                        acc[q] = acc[q] + cm * crows_v[i0 + m, pl.ds(q * L, L)]
                dot = zeros
                for q in range(NQ):
                    dot = dot + acc[q] * wemb_v[r, pl.ds(q * L, L)]
                o0 = r * (8 * L) + j * L
                ips_v[pl.ds(o0, L)] = -dot if j == 0 else dot
                if j == 0:
                    ips_v[pl.ds(r * (8 * L) + 6 * L, L)] = zeros
                    ips_v[pl.ds(r * (8 * L) + 7 * L, L)] = zeros
                return c2

            lax.fori_loop(0, C, crow, 0)

        pltpu.sync_copy(ips_v, out_hbm.at[pl.ds(base * 8 * L, C * 8 * L)])
        return carry

    lax.fori_loop(0, NCHUNK, chunk_body, 0)


_sc_ips = pl.kernel(
    _sc_body,
    out_type=jax.ShapeDtypeStruct((B * 8 * L,), jnp.float32),
    mesh=plsc.VectorSubcoreMesh(core_axis_name="c", subcore_axis_name="s"),
    compiler_params=pltpu.CompilerParams(use_tc_tiling_on_sc=False),
    scratch_types=[
        pltpu.VMEM((GPC, IDXG), jnp.int32),
        pltpu.VMEM((C * M + L,), jnp.float32),
        pltpu.VMEM((C * M, SIZE), jnp.float32),
        pltpu.VMEM((GPC, IDXG), jnp.int32),
        pltpu.VMEM((C * M + L,), jnp.float32),
        pltpu.VMEM((C * M, SIZE), jnp.float32),
        pltpu.VMEM((C, SIZE), jnp.float32),
        pltpu.VMEM((C * 8 * L,), jnp.float32),
        pltpu.SemaphoreType.DMA,
    ],
)


def _loss_body(x_ref, w_ref, o_ref):
    i = lax.broadcasted_iota(jnp.int32, (128, 8), 0)
    j = lax.broadcasted_iota(jnp.int32, (128, 8), 1)
    g = jnp.where(i // L == j, 1.0, 0.0).astype(jnp.float32)
    y = jnp.dot(x_ref[...], g, preferred_element_type=jnp.float32)
    y = jnp.clip(y, -10.0, 10.0)
    part = jnp.sum(w_ref[...] * jax.nn.softplus(y))

    @pl.when(pl.program_id(0) == 0)
    def _():
        o_ref[...] = jnp.zeros_like(o_ref)

    o_ref[...] = o_ref[...] + jnp.full((1, 1), part, jnp.float32)


def _loss_tc(x2d, w2d):
    grid = (B // TC_ROWS,)
    return pl.pallas_call(
        _loss_body,
        grid=grid,
        in_specs=[
            pl.BlockSpec((TC_ROWS, 128), lambda i: (i, 0)),
            pl.BlockSpec((TC_ROWS, 8), lambda i: (i, 0)),
        ],
        out_specs=pl.BlockSpec((1, 1), lambda i: (0, 0)),
        out_shape=jax.ShapeDtypeStruct((1, 1), jnp.float32),
    )(x2d, w2d)


def kernel(data, word2morph, word2morph_mask, ctx2morph, ctx2morph_mask, emb0, emb1):
    w2m_g = word2morph.reshape(NW * NCHUNK, GPC, IDXG)
    wmask = word2morph_mask.reshape(B * M)
    c2m_g = jnp.transpose(ctx2morph, (1, 0, 2)).reshape(
        NSLOT * NW * NCHUNK, GPC, IDXG)
    cmask = jnp.transpose(ctx2morph_mask[..., 0], (1, 0, 2)).reshape(NSLOT * B * M)

    ips = _sc_ips(w2m_g, wmask, c2m_g, cmask, emb0, emb1)

    neg_mask = data[:, 2 + NEG:].astype(jnp.float32)
    wts = jnp.concatenate(
        [jnp.ones((B, 1), jnp.float32), neg_mask, jnp.zeros((B, 2), jnp.float32)],
        axis=1)

    loss = _loss_tc(ips.reshape(B, 8 * L), wts)
    return loss[0, 0]


# speedup vs baseline: 6.5536x; 1.4209x over previous
import jax
import jax.numpy as jnp
from jax import lax
from jax.experimental import pallas as pl
from jax.experimental.pallas import tpu as pltpu
from jax.experimental.pallas import tpu_sc as plsc

B = 16384
SIZE = 64
M = 5
NEG = 5
NSLOT = 1 + NEG

NC = 2
NS = 16
NW = NC * NS
L = 16

ROWS_PER_W = B // NW
C = 64
NCHUNK = ROWS_PER_W // C
IDXG = 64
GPC = C * M // IDXG
NQ = SIZE // L

RWM = ROWS_PER_W * M
RCM = NSLOT * RWM

TC_ROWS = 2048


def _sc_body(w2m_hbm, wmask_hbm, c2m_hbm, cmask_hbm, emb0_hbm, emb1_hbm,
             out_hbm,
             widx_all, cidx_all, wmask_all, cmask_all, rows0, rows1, wemb_v,
             ips_v, sem0, sem1):
    wid = lax.axis_index("s") * NC + lax.axis_index("c")
    zeros = jnp.zeros((L,), jnp.float32)

    pltpu.sync_copy(w2m_hbm.at[wid], widx_all)
    pltpu.sync_copy(c2m_hbm.at[wid], cidx_all)
    pltpu.sync_copy(wmask_hbm.at[pl.ds(wid * RWM, RWM)],
                    wmask_all.at[pl.ds(0, RWM)])
    pltpu.sync_copy(cmask_hbm.at[pl.ds(wid * RCM, RCM)],
                    cmask_all.at[pl.ds(0, RCM)])

    def issue_round(ch, r, rows_v, sem):
        table = emb0_hbm if r == 0 else emb1_hbm
        idx = widx_all if r == 0 else cidx_all
        row0 = ch * GPC if r == 0 else (r - 1) * (NCHUNK * GPC) + ch * GPC
        for g in range(GPC):
            pltpu.async_copy(table.at[idx.at[row0 + g]],
                             rows_v.at[pl.ds(g * IDXG, IDXG)], sem)

    def drain(rows_v, sem):
        pltpu.make_async_copy(emb0_hbm.at[pl.ds(0, C * M)], rows_v, sem).wait()

    def compute_wpool(ch, rows_v):
        moff = ch * (C * M)

        def body(r, c2):
            i0 = r * M
            mvec = wmask_all[pl.ds(moff + i0, L)]
            acc = [zeros for _ in range(NQ)]
            for m in range(M):
                wm = mvec[m]
                for q in range(NQ):
                    acc[q] = acc[q] + wm * rows_v[i0 + m, pl.ds(q * L, L)]
            for q in range(NQ):
                wemb_v[r, pl.ds(q * L, L)] = acc[q]
            return c2

        lax.fori_loop(0, C, body, 0)

    def compute_slot(ch, j, rows_v):
        moff = j * (NCHUNK * C * M) + ch * (C * M)

        def body(r, c2, j=j):
            i0 = r * M
            mvec = cmask_all[pl.ds(moff + i0, L)]
            wq = [wemb_v[r, pl.ds(q * L, L)] for q in range(NQ)]
            acc = zeros
            for m in range(M):
                pm = rows_v[i0 + m, pl.ds(0, L)] * wq[0]
                for q in range(1, NQ):
                    pm = pm + rows_v[i0 + m, pl.ds(q * L, L)] * wq[q]
                acc = acc + mvec[m] * pm
            o0 = r * (8 * L) + j * L
            ips_v[pl.ds(o0, L)] = -acc if j == 0 else acc
            if j == 0:
                ips_v[pl.ds(r * (8 * L) + 6 * L, L)] = zeros
                ips_v[pl.ds(r * (8 * L) + 7 * L, L)] = zeros
            return c2

        lax.fori_loop(0, C, body, 0)

    issue_round(0, 0, rows0, sem0)

    def pair_body(i, carry):
        for half in range(2):
            ch = i * 2 + half
            for r in range(NSLOT + 1):
                par = (half + r) % 2
                rows_cur, sem_cur = (rows0, sem0) if par == 0 else (rows1, sem1)
                rows_nxt, sem_nxt = (rows1, sem1) if par == 0 else (rows0, sem0)
                if r < NSLOT:
                    issue_round(ch, r + 1, rows_nxt, sem_nxt)
                else:
                    chn = ch + 1

                    @pl.when(chn < NCHUNK)
                    def _():
                        issue_round(chn, 0, rows_nxt, sem_nxt)

                drain(rows_cur, sem_cur)
                if r == 0:
                    compute_wpool(ch, rows_cur)
                else:
                    compute_slot(ch, r - 1, rows_cur)
            base = (wid * NCHUNK + ch) * C
            pltpu.sync_copy(ips_v, out_hbm.at[pl.ds(base * 8 * L, C * 8 * L)])
        return carry

    lax.fori_loop(0, NCHUNK // 2, pair_body, 0)


_sc_ips = pl.kernel(
    _sc_body,
    out_type=jax.ShapeDtypeStruct((B * 8 * L,), jnp.float32),
    mesh=plsc.VectorSubcoreMesh(core_axis_name="c", subcore_axis_name="s"),
    compiler_params=pltpu.CompilerParams(use_tc_tiling_on_sc=False),
    scratch_types=[
        pltpu.VMEM((NCHUNK * GPC, IDXG), jnp.int32),
        pltpu.VMEM((NSLOT * NCHUNK * GPC, IDXG), jnp.int32),
        pltpu.VMEM((RWM + L,), jnp.float32),
        pltpu.VMEM((RCM + L,), jnp.float32),
        pltpu.VMEM((C * M, SIZE), jnp.float32),
        pltpu.VMEM((C * M, SIZE), jnp.float32),
        pltpu.VMEM((C, SIZE), jnp.float32),
        pltpu.VMEM((C * 8 * L,), jnp.float32),
        pltpu.SemaphoreType.DMA,
        pltpu.SemaphoreType.DMA,
    ],
)


def _loss_body(x_ref, w_ref, o_ref):
    i = lax.broadcasted_iota(jnp.int32, (128, 8), 0)
    j = lax.broadcasted_iota(jnp.int32, (128, 8), 1)
    g = jnp.where(i // L == j, 1.0, 0.0).astype(jnp.float32)
    y = jnp.dot(x_ref[...], g, preferred_element_type=jnp.float32)
    y = jnp.clip(y, -10.0, 10.0)
    part = jnp.sum(w_ref[...] * jax.nn.softplus(y))

    @pl.when(pl.program_id(0) == 0)
    def _():
        o_ref[...] = jnp.zeros_like(o_ref)

    o_ref[...] = o_ref[...] + jnp.full((1, 1), part, jnp.float32)


def _loss_tc(x2d, w2d):
    grid = (B // TC_ROWS,)
    return pl.pallas_call(
        _loss_body,
        grid=grid,
        in_specs=[
            pl.BlockSpec((TC_ROWS, 128), lambda i: (i, 0)),
            pl.BlockSpec((TC_ROWS, 8), lambda i: (i, 0)),
        ],
        out_specs=pl.BlockSpec((1, 1), lambda i: (0, 0)),
        out_shape=jax.ShapeDtypeStruct((1, 1), jnp.float32),
    )(x2d, w2d)


def kernel(data, word2morph, word2morph_mask, ctx2morph, ctx2morph_mask, emb0, emb1):
    w2m_g = word2morph.reshape(NW, NCHUNK * GPC, IDXG)
    wmask = word2morph_mask.reshape(B * M)
    c2m_g = jnp.transpose(
        ctx2morph.reshape(NW, ROWS_PER_W, NSLOT, M), (0, 2, 1, 3)
    ).reshape(NW, NSLOT * NCHUNK * GPC, IDXG)
    cmask = jnp.transpose(
        ctx2morph_mask[..., 0].reshape(NW, ROWS_PER_W, NSLOT, M), (0, 2, 1, 3)
    ).reshape(NW * RCM)

    ips = _sc_ips(w2m_g, wmask, c2m_g, cmask, emb0, emb1)

    neg_mask = data[:, 2 + NEG:].astype(jnp.float32)
    wts = jnp.concatenate(
        [jnp.ones((B, 1), jnp.float32), neg_mask, jnp.zeros((B, 2), jnp.float32)],
        axis=1)

    loss = _loss_tc(ips.reshape(B, 8 * L), wts)
    return loss[0, 0]
